# baseline (speedup 1.0000x reference)
import functools

import jax
import jax.numpy as jnp
from jax import lax
from jax.experimental import pallas as pl
from jax.experimental.pallas import tpu as pltpu
from jax.experimental.pallas import tpu_sc as plsc

N = 10000
NP = 10240
E = 160000
B = 512
H = 2
OC = 128
D = H * OC
ED = 64
V = 5
T = 21
EPS = 1e-5

EP = E // 16
CH = 80
NCH = EP // CH
NPT = NP // 16
SCALE = 1.0 / (OC ** 0.5)


def _edge_body(qh, kh, vh, srch, dsth, out, alout,
               qidx, kidx, dstb, tmp80, arow, qbuf, kbuf, exrow,
               den, dacc, dtmp, vmax16, mbuf, aggbuf,
               agg_s, dens_s, maxes_s, sem0, sem1):
    c = lax.axis_index("c")
    w = lax.axis_index("s")
    ebase = w * EP
    nbase = w * NPT
    zero16 = jnp.zeros((16,), jnp.float32)
    izero = jnp.zeros((16,), jnp.int32)
    lane = lax.iota(jnp.int32, 16)

    def bcast_lane(vec, r):
        dnums = lax.GatherDimensionNumbers(
            offset_dims=(), collapsed_slice_dims=(0,), start_index_map=(0,))
        return lax.gather(vec, (izero + r)[:, None], dnums, (1,),
                          mode=lax.GatherScatterMode.PROMISE_IN_BOUNDS)

    def z_qbuf(r, _):
        for jj in range(8):
            qbuf[r, pl.ds(jj * 16, 16)] = zero16
        return 0
    lax.fori_loop(0, CH, z_qbuf, 0)
    for t in range(NPT // CH):
        pltpu.sync_copy(qbuf, agg_s.at[pl.ds(pl.multiple_of(nbase + t * CH, 16), CH)])

    def z_den(i, _):
        den[pl.ds(i * 16, 16)] = zero16
        return 0
    lax.fori_loop(0, NP // 16, z_den, 0)

    def p1_chunk(j, runmax):
        eb = ebase + j * CH
        pltpu.sync_copy(dsth.at[pl.ds(eb, CH)], tmp80)

        def idx_d(i, _):
            d16 = tmp80[pl.ds(i * 16, 16)]
            qidx[pl.ds(i * 16, 16)] = d16 * 2 + c
            return 0
        lax.fori_loop(0, CH // 16, idx_d, 0)
        pltpu.sync_copy(srch.at[pl.ds(eb, CH)], tmp80)

        def idx_s(i, _):
            s16 = tmp80[pl.ds(i * 16, 16)]
            kidx[pl.ds(i * 16, 16)] = s16 * 2 + c
            return 0
        lax.fori_loop(0, CH // 16, idx_s, 0)

        cq = pltpu.async_copy(qh.at[qidx], qbuf, sem0)
        ck = pltpu.async_copy(kh.at[kidx], kbuf, sem1)
        cq.wait()
        ck.wait()

        def grp(g, rm):
            def edg(e16, carry):
                a16, rmi = carry
                e = g * 16 + e16
                acc = qbuf[e, pl.ds(0, 16)] * kbuf[e, pl.ds(0, 16)]
                for jj in range(1, 8):
                    acc = acc + (qbuf[e, pl.ds(jj * 16, 16)]
                                 * kbuf[e, pl.ds(jj * 16, 16)])
                aval = jnp.sum(acc) * SCALE
                a16 = jnp.where(lane == e16, aval, a16)
                return (a16, rmi)
            a16, rm = lax.fori_loop(0, 16, edg, (zero16, rm))
            arow[pl.ds(g * 16, 16)] = a16
            return jnp.maximum(rm, a16)
        runmax = lax.fori_loop(0, CH // 16, grp, runmax)
        pltpu.sync_copy(arow, alout.at[pl.ds(pl.multiple_of(c * E + eb, 16), CH)])
        return runmax

    runmax = lax.fori_loop(0, NCH, p1_chunk,
                           jnp.full((16,), -1e30, jnp.float32))
    vmax16[pl.ds(0, 16)] = runmax
    pltpu.sync_copy(vmax16, maxes_s.at[pl.ds(pl.multiple_of(w * 16, 16), 16)])
    plsc.subcore_barrier()

    pltpu.sync_copy(maxes_s, mbuf)
    m = mbuf[pl.ds(0, 16)]
    for t in range(1, 16):
        m = jnp.maximum(m, mbuf[pl.ds(t * 16, 16)])
    gmax = jnp.max(m)

    def p2_chunk(j, _):
        eb = ebase + j * CH
        pltpu.sync_copy(srch.at[pl.ds(eb, CH)], tmp80)

        def idx_s(i, _):
            s16 = tmp80[pl.ds(i * 16, 16)]
            kidx[pl.ds(i * 16, 16)] = s16 * 2 + c
            return 0
        lax.fori_loop(0, CH // 16, idx_s, 0)
        cv = pltpu.async_copy(vh.at[kidx], qbuf, sem0)
        pltpu.sync_copy(dsth.at[pl.ds(eb, CH)], tmp80)

        def idx_d(i, _):
            dstb[0, pl.ds(i * 16, 16)] = tmp80[pl.ds(i * 16, 16)]
            return 0
        lax.fori_loop(0, CH // 16, idx_d, 0)
        pltpu.sync_copy(alout.at[pl.ds(pl.multiple_of(c * E + eb, 16), CH)], arow)

        def exg(i, _):
            a16 = arow[pl.ds(i * 16, 16)]
            e16 = jnp.exp(a16 - gmax)
            exrow[pl.ds(i * 16, 16)] = e16
            d16 = dstb[0, pl.ds(i * 16, 16)]
            plsc.addupdate_scatter(den, [d16], e16)
            return 0
        lax.fori_loop(0, CH // 16, exg, 0)
        cv.wait()

        def sgrp(g, _):
            ex16 = exrow[pl.ds(g * 16, 16)]

            def srow(r, _):
                e = g * 16 + r
                bc = bcast_lane(ex16, r)
                for jj in range(8):
                    qbuf[e, pl.ds(jj * 16, 16)] = (
                        qbuf[e, pl.ds(jj * 16, 16)] * bc)
                return 0
            lax.fori_loop(0, 16, srow, 0)
            return 0
        lax.fori_loop(0, CH // 16, sgrp, 0)
        pltpu.sync_copy(qbuf, agg_s.at[dstb.at[0]], add=True)
        return 0
    lax.fori_loop(0, NCH, p2_chunk, 0)
    pltpu.sync_copy(den, dens_s.at[pl.ds(pl.multiple_of(w * NP, 16), NP)])
    plsc.subcore_barrier()

    def dz(i, _):
        dacc[pl.ds(i * 16, 16)] = zero16
        return 0
    lax.fori_loop(0, NPT // 16, dz, 0)

    def dred(t, _):
        pltpu.sync_copy(dens_s.at[pl.ds(pl.multiple_of(t * NP + nbase, 16), NPT)], dtmp)

        def dadd(i, _):
            dacc[pl.ds(i * 16, 16)] = (dacc[pl.ds(i * 16, 16)]
                                       + dtmp[pl.ds(i * 16, 16)])
            return 0
        lax.fori_loop(0, NPT // 16, dadd, 0)
        return 0
    lax.fori_loop(0, 16, dred, 0)

    def outc(ch, _):
        rb = nbase + ch * 16
        pltpu.sync_copy(agg_s.at[pl.ds(pl.multiple_of(rb, 16), 16)], aggbuf)

        inv16 = 1.0 / (dacc[pl.ds(ch * 16, 16)] + 1e-16)

        def rown(r, _):
            bc = bcast_lane(inv16, r)
            for jj in range(8):
                aggbuf[r, pl.ds(jj * 16, 16)] = (
                    aggbuf[r, pl.ds(jj * 16, 16)] * bc)
            return 0
        lax.fori_loop(0, 16, rown, 0)
        pltpu.sync_copy(aggbuf, out.at[pl.ds(pl.multiple_of(c * NP + rb, 16), 16)])
        return 0
    lax.fori_loop(0, NPT // 16, outc, 0)


def _edge_call(qh, kh, vh, srcv, dstv):
    mesh = plsc.VectorSubcoreMesh(core_axis_name="c", subcore_axis_name="s")
    f = pl.kernel(
        _edge_body,
        out_type=(jax.ShapeDtypeStruct((H * NP, OC), jnp.float32),
                  jax.ShapeDtypeStruct((H * E,), jnp.float32)),
        mesh=mesh,
        compiler_params=pltpu.CompilerParams(needs_layout_passes=False),
        scratch_types=[
            pltpu.VMEM((CH,), jnp.int32),
            pltpu.VMEM((CH,), jnp.int32),
            pltpu.VMEM((1, CH), jnp.int32),
            pltpu.VMEM((CH,), jnp.int32),
            pltpu.VMEM((CH,), jnp.float32),
            pltpu.VMEM((CH, OC), jnp.float32),
            pltpu.VMEM((CH, OC), jnp.float32),
            pltpu.VMEM((CH,), jnp.float32),
            pltpu.VMEM((NP,), jnp.float32),
            pltpu.VMEM((NPT,), jnp.float32),
            pltpu.VMEM((NPT,), jnp.float32),
            pltpu.VMEM((16,), jnp.float32),
            pltpu.VMEM((256,), jnp.float32),
            pltpu.VMEM((16, OC), jnp.float32),
            pltpu.VMEM_SHARED((NP, OC), jnp.float32),
            pltpu.VMEM_SHARED((16 * NP,), jnp.float32),
            pltpu.VMEM_SHARED((256,), jnp.float32),
            pltpu.SemaphoreType.DMA,
            pltpu.SemaphoreType.DMA,
        ],
    )
    agg, _unused_alpha = f(qh, kh, vh, srcv, dstv)
    return agg


PB = 768
NH = NP // 2


def _qkvs_body(h_ref, wq_ref, wk_ref, wv_ref, ws_ref,
               bq_ref, bk_ref, bv_ref, bs_ref,
               q_ref, k_ref, v_ref, s_ref):
    hb = h_ref[...]
    q_ref[...] = jnp.dot(hb, wq_ref[...],
                         preferred_element_type=jnp.float32) + bq_ref[...]
    k_ref[...] = jnp.dot(hb, wk_ref[...],
                         preferred_element_type=jnp.float32) + bk_ref[...]
    v_ref[...] = jnp.dot(hb, wv_ref[...],
                         preferred_element_type=jnp.float32) + bv_ref[...]
    s_ref[...] = jnp.dot(hb, ws_ref[...],
                         preferred_element_type=jnp.float32) + bs_ref[...]


def _qkvs(h, Wq, Wk, Wv, Ws, bq, bk, bv, bs):
    cin = h.shape[1]
    wspec = pl.BlockSpec((cin, D), lambda i: (0, 0))
    bspec = pl.BlockSpec((1, D), lambda i: (0, 0))
    ospec = pl.BlockSpec((512, D), lambda i: (i, 0))
    outs = pl.pallas_call(
        _qkvs_body,
        grid=(NP // 512,),
        in_specs=[pl.BlockSpec((512, cin), lambda i: (i, 0)),
                  wspec, wspec, wspec, wspec,
                  bspec, bspec, bspec, bspec],
        out_specs=[ospec, ospec, ospec, ospec],
        out_shape=[jax.ShapeDtypeStruct((NP, D), jnp.float32)] * 4,
    )(h, Wq, Wk, Wv, Ws, bq.reshape(1, D), bk.reshape(1, D),
      bv.reshape(1, D), bs.reshape(1, D))
    return outs


def _postln_body(a0_ref, a1_ref, s_ref, g_ref, b_ref, h_ref):
    xx = jnp.concatenate([a0_ref[...], a1_ref[...]], axis=1) + s_ref[...]
    xx = jax.nn.relu(xx)
    mu = jnp.mean(xx, axis=-1, keepdims=True)
    var = jnp.mean(jnp.square(xx - mu), axis=-1, keepdims=True)
    h_ref[...] = (xx - mu) / jnp.sqrt(var + EPS) * g_ref[...] + b_ref[...]


def _postln(agg, s, g, b):
    return pl.pallas_call(
        _postln_body,
        grid=(NP // 512,),
        in_specs=[pl.BlockSpec((512, OC), lambda i: (i, 0)),
                  pl.BlockSpec((512, OC), lambda i: (i + NP // 512, 0)),
                  pl.BlockSpec((512, D), lambda i: (i, 0)),
                  pl.BlockSpec((1, D), lambda i: (0, 0)),
                  pl.BlockSpec((1, D), lambda i: (0, 0))],
        out_specs=pl.BlockSpec((512, D), lambda i: (i, 0)),
        out_shape=jax.ShapeDtypeStruct((NP, D), jnp.float32),
    )(agg, agg, s, g.reshape(1, D), b.reshape(1, D))


def _tconv_sc(h, srcv, dstv, Wq, bq, Wk, bk, Wv, bv, Ws, bs, g, b):
    q, k, v, s = _qkvs(h, Wq, Wk, Wv, Ws, bq, bk, bv, bs)
    agg = _edge_call(q.reshape(2 * NP, OC), k.reshape(2 * NP, OC),
                     v.reshape(2 * NP, OC), srcv, dstv)
    return _postln(agg, s, g, b)


def _decoder_body(pooled_ref, gc_ref, sl_ref, fcWp_ref, fcg_ref,
                  fcs_ref, fcb_ref, emb_ref, Wih_ref, Whh_ref,
                  bihh_ref, outW_ref, outb_ref, out_ref):
    pooled = pooled_ref[...]
    enc = jax.nn.relu(
        jnp.dot(pooled, fcWp_ref[...], preferred_element_type=jnp.float32)
        + jnp.dot(gc_ref[...], fcg_ref[...],
                  preferred_element_type=jnp.float32)
        + jnp.dot(sl_ref[...], fcs_ref[...],
                  preferred_element_type=jnp.float32)
        + fcb_ref[...])
    emb = emb_ref[...]
    Wih = Wih_ref[...]
    Whh = Whh_ref[...]
    bihh = bihh_ref[...]
    outW = outW_ref[...]
    outb = outb_ref[...]
    hs = enc
    cs = jnp.zeros_like(enc)
    inp = jnp.broadcast_to(emb[1], (B, ED))
    for t in range(T):
        gates = (jnp.dot(inp, Wih, preferred_element_type=jnp.float32)
                 + jnp.dot(hs, Whh, preferred_element_type=jnp.float32)
                 + bihh)
        i_ = gates[:, 0 * ED:1 * ED]
        f_ = gates[:, 1 * ED:2 * ED]
        g_ = gates[:, 2 * ED:3 * ED]
        o_ = gates[:, 3 * ED:4 * ED]
        cs = jax.nn.sigmoid(f_) * cs + jax.nn.sigmoid(i_) * jnp.tanh(g_)
        hs = jax.nn.sigmoid(o_) * jnp.tanh(cs)
        logits = (jnp.dot(hs, outW, preferred_element_type=jnp.float32)
                  + outb)
        out_ref[:, t, :] = logits
        m = jnp.max(logits, axis=-1, keepdims=True)
        iota = lax.broadcasted_iota(jnp.int32, (B, V), 1)
        tok = jnp.min(jnp.where(logits == m, iota, V), axis=-1,
                      keepdims=True)
        onehot = (lax.broadcasted_iota(jnp.int32, (B, V), 1)
                  == tok).astype(jnp.float32)
        inp = jnp.dot(onehot, emb, preferred_element_type=jnp.float32)


def _decode(pooled, gc, sl, fcWp, fcg, fcs, fcb, emb, Wih, Whh, bihh,
            outW, outb):
    return pl.pallas_call(
        _decoder_body,
        out_shape=jax.ShapeDtypeStruct((B, T, V), jnp.float32),
    )(pooled, gc, sl, fcWp, fcg, fcs, fcb, emb, Wih, Whh, bihh,
      outW, outb)


def kernel(x, edge_index, batch, gc_content, seq_length,
           Wq0, bq0, Wk0, bk0, Wv0, bv0, Ws0, bs0, g0, b0,
           Wq1, bq1, Wk1, bk1, Wv1, bv1, Ws1, bs1, g1, b1,
           Wq2, bq2, Wk2, bk2, Wv2, bv2, Ws2, bs2, g2, b2,
           fcW, fcb, emb, Wih, Whh, bih, bhh, outW, outb):
    srcv = edge_index[0]
    dstv = edge_index[1]
    xp = jnp.zeros((NP, 4), jnp.float32).at[:N].set(x)
    batch_pad = jnp.concatenate(
        [batch, jnp.full((NP - N,), B, jnp.int32)])
    layers = [
        (Wq0, bq0, Wk0, bk0, Wv0, bv0, Ws0, bs0, g0, b0),
        (Wq1, bq1, Wk1, bk1, Wv1, bv1, Ws1, bs1, g1, b1),
        (Wq2, bq2, Wk2, bk2, Wv2, bv2, Ws2, bs2, g2, b2),
    ]
    h = xp
    for (Wq, bq, Wk, bk, Wv, bv, Ws, bs, g, b) in layers:
        h = _tconv_sc(h, srcv, dstv, Wq, bq, Wk, bk, Wv, bv, Ws, bs, g, b)
    hn = h[:N]
    cnt = jax.ops.segment_sum(jnp.ones((N,), jnp.float32), batch,
                              num_segments=B)
    pooled = (jax.ops.segment_sum(hn, batch, num_segments=B)
              / jnp.maximum(cnt, 1.0)[:, None])
    return _decode(pooled,
                   gc_content.reshape(B, 1), seq_length.reshape(B, 1),
                   fcW[:D], fcW[D:D + 1], fcW[D + 1:D + 2], fcb.reshape(1, ED),
                   emb, Wih, Whh, (bih + bhh).reshape(1, 4 * ED), outW,
                   outb.reshape(1, V))


# speedup vs baseline: 1.0131x; 1.0131x over previous
import functools

import jax
import jax.numpy as jnp
from jax import lax
from jax.experimental import pallas as pl
from jax.experimental.pallas import tpu as pltpu
from jax.experimental.pallas import tpu_sc as plsc

N = 10000
NP = 10240
E = 160000
B = 512
H = 2
OC = 128
D = H * OC
ED = 64
V = 5
T = 21
EPS = 1e-5

EP = E // 16
CH = 80
NCH = EP // CH
NPT = NP // 16
SCALE = 1.0 / (OC ** 0.5)


def _edge_body(qh, kh, vh, srch, dsth, out, alout,
               qidx, kidx, dstb, tmp80, arow, qbuf, kbuf, exrow,
               den2, dtmp2, iob, vmax16, mbuf, aggbuf,
               agg_s, den_s, maxes_s, sem0, sem1):
    c = lax.axis_index("c")
    w = lax.axis_index("s")
    ebase = w * EP
    nbase = w * NPT
    zero16 = jnp.zeros((16,), jnp.float32)
    izero = jnp.zeros((16,), jnp.int32)
    lane = lax.iota(jnp.int32, 16)

    dnums = lax.GatherDimensionNumbers(
        offset_dims=(), collapsed_slice_dims=(0,), start_index_map=(0,))

    def permute(vec, idxv):
        return lax.gather(vec, idxv[:, None], dnums, (1,),
                          mode=lax.GatherScatterMode.PROMISE_IN_BOUNDS)

    def bcast_lane(vec, r):
        return permute(vec, izero + r)

    bfly = [lane ^ k for k in (8, 4, 2, 1)]

    def lanesum(vec):
        for ix in bfly:
            vec = vec + permute(vec, ix)
        return vec

    def z_qbuf(r, _):
        for jj in range(8):
            qbuf[r, pl.ds(jj * 16, 16)] = zero16
        return 0
    lax.fori_loop(0, CH, z_qbuf, 0)
    for t in range(NPT // CH):
        pltpu.sync_copy(qbuf, agg_s.at[pl.ds(pl.multiple_of(nbase + t * CH, 16), CH)])

    def z_den(i, _):
        for jj in range(8):
            den2[i, pl.ds(jj * 16, 16)] = zero16
        return 0
    lax.fori_loop(0, NP // 128, z_den, 0)

    def z_dt(i, _):
        for jj in range(8):
            dtmp2[i, pl.ds(jj * 16, 16)] = zero16
        return 0
    lax.fori_loop(0, 5, z_dt, 0)
    pltpu.sync_copy(dtmp2, den_s.at[pl.ds(pl.multiple_of(w * 5, 1), 5)])
    for i in range(5):
        iob[0, pl.ds(i * 16, 16)] = i * 16 + lane

    def p1_chunk(j, runmax):
        eb = ebase + j * CH
        pltpu.sync_copy(dsth.at[pl.ds(eb, CH)], tmp80)

        def idx_d(i, _):
            d16 = tmp80[pl.ds(i * 16, 16)]
            qidx[pl.ds(i * 16, 16)] = d16 * 2 + c
            return 0
        lax.fori_loop(0, CH // 16, idx_d, 0)
        pltpu.sync_copy(srch.at[pl.ds(eb, CH)], tmp80)

        def idx_s(i, _):
            s16 = tmp80[pl.ds(i * 16, 16)]
            kidx[pl.ds(i * 16, 16)] = s16 * 2 + c
            return 0
        lax.fori_loop(0, CH // 16, idx_s, 0)

        cq = pltpu.async_copy(qh.at[qidx], qbuf, sem0)
        ck = pltpu.async_copy(kh.at[kidx], kbuf, sem1)
        cq.wait()
        ck.wait()

        def grp(g, rm):
            def sub(s4, a16):
                for u in range(4):
                    e16 = s4 * 4 + u
                    e = g * 16 + e16
                    acc = qbuf[e, pl.ds(0, 16)] * kbuf[e, pl.ds(0, 16)]
                    for jj in range(1, 8):
                        acc = acc + (qbuf[e, pl.ds(jj * 16, 16)]
                                     * kbuf[e, pl.ds(jj * 16, 16)])
                    aval = lanesum(acc)
                    a16 = jnp.where(lane == e16, aval, a16)
                return a16
            a16 = lax.fori_loop(0, 4, sub, zero16) * SCALE
            arow[pl.ds(g * 16, 16)] = a16
            return jnp.maximum(rm, a16)
        runmax = lax.fori_loop(0, CH // 16, grp, runmax)
        pltpu.sync_copy(arow, alout.at[pl.ds(pl.multiple_of(c * E + eb, 16), CH)])
        return runmax

    runmax = lax.fori_loop(0, NCH, p1_chunk,
                           jnp.full((16,), -1e30, jnp.float32))
    vmax16[pl.ds(0, 16)] = runmax
    pltpu.sync_copy(vmax16, maxes_s.at[pl.ds(pl.multiple_of(w * 16, 16), 16)])
    plsc.subcore_barrier()

    pltpu.sync_copy(maxes_s, mbuf)
    m = mbuf[pl.ds(0, 16)]
    for t in range(1, 16):
        m = jnp.maximum(m, mbuf[pl.ds(t * 16, 16)])
    gmax = jnp.max(m)

    def p2_chunk(j, _):
        eb = ebase + j * CH
        pltpu.sync_copy(srch.at[pl.ds(eb, CH)], tmp80)

        def idx_s(i, _):
            s16 = tmp80[pl.ds(i * 16, 16)]
            kidx[pl.ds(i * 16, 16)] = s16 * 2 + c
            return 0
        lax.fori_loop(0, CH // 16, idx_s, 0)
        cv = pltpu.async_copy(vh.at[kidx], qbuf, sem0)
        pltpu.sync_copy(dsth.at[pl.ds(eb, CH)], tmp80)

        def idx_d(i, _):
            dstb[0, pl.ds(i * 16, 16)] = tmp80[pl.ds(i * 16, 16)]
            return 0
        lax.fori_loop(0, CH // 16, idx_d, 0)
        pltpu.sync_copy(alout.at[pl.ds(pl.multiple_of(c * E + eb, 16), CH)], arow)

        def exg(i, _):
            a16 = arow[pl.ds(i * 16, 16)]
            e16 = jnp.exp(a16 - gmax)
            exrow[pl.ds(i * 16, 16)] = e16
            d16 = dstb[0, pl.ds(i * 16, 16)]
            plsc.addupdate_scatter(den2, [d16 >> 7, d16 & 127], e16)
            return 0
        lax.fori_loop(0, CH // 16, exg, 0)
        cv.wait()

        def sgrp(g, _):
            ex16 = exrow[pl.ds(g * 16, 16)]

            def ssub(s4, _):
                for u in range(4):
                    r = s4 * 4 + u
                    e = g * 16 + r
                    bc = bcast_lane(ex16, r)
                    for jj in range(8):
                        qbuf[e, pl.ds(jj * 16, 16)] = (
                            qbuf[e, pl.ds(jj * 16, 16)] * bc)
                return 0
            lax.fori_loop(0, 4, ssub, 0)
            return 0
        lax.fori_loop(0, CH // 16, sgrp, 0)
        pltpu.sync_copy(qbuf, agg_s.at[dstb.at[0]], add=True)
        return 0
    lax.fori_loop(0, NCH, p2_chunk, 0)
    pltpu.sync_copy(den2, den_s.at[iob.at[0]], add=True)
    plsc.subcore_barrier()

    pltpu.sync_copy(den_s.at[pl.ds(pl.multiple_of(w * 5, 1), 5)], dtmp2)

    def outc(ch, _):
        rb = nbase + ch * 16
        pltpu.sync_copy(agg_s.at[pl.ds(pl.multiple_of(rb, 16), 16)], aggbuf)

        inv16 = 1.0 / (dtmp2[ch >> 3, pl.ds((ch & 7) * 16, 16)] + 1e-16)

        def rown(r, _):
            bc = bcast_lane(inv16, r)
            for jj in range(8):
                aggbuf[r, pl.ds(jj * 16, 16)] = (
                    aggbuf[r, pl.ds(jj * 16, 16)] * bc)
            return 0
        lax.fori_loop(0, 16, rown, 0)
        pltpu.sync_copy(aggbuf, out.at[pl.ds(pl.multiple_of(c * NP + rb, 16), 16)])
        return 0
    lax.fori_loop(0, NPT // 16, outc, 0)


def _edge_call(qh, kh, vh, srcv, dstv):
    mesh = plsc.VectorSubcoreMesh(core_axis_name="c", subcore_axis_name="s")
    f = pl.kernel(
        _edge_body,
        out_type=(jax.ShapeDtypeStruct((H * NP, OC), jnp.float32),
                  jax.ShapeDtypeStruct((H * E,), jnp.float32)),
        mesh=mesh,
        compiler_params=pltpu.CompilerParams(needs_layout_passes=False),
        scratch_types=[
            pltpu.VMEM((CH,), jnp.int32),
            pltpu.VMEM((CH,), jnp.int32),
            pltpu.VMEM((1, CH), jnp.int32),
            pltpu.VMEM((CH,), jnp.int32),
            pltpu.VMEM((CH,), jnp.float32),
            pltpu.VMEM((CH, OC), jnp.float32),
            pltpu.VMEM((CH, OC), jnp.float32),
            pltpu.VMEM((CH,), jnp.float32),
            pltpu.VMEM((NP // 128, 128), jnp.float32),
            pltpu.VMEM((5, 128), jnp.float32),
            pltpu.VMEM((1, 80), jnp.int32),
            pltpu.VMEM((16,), jnp.float32),
            pltpu.VMEM((256,), jnp.float32),
            pltpu.VMEM((16, OC), jnp.float32),
            pltpu.VMEM_SHARED((NP, OC), jnp.float32),
            pltpu.VMEM_SHARED((NP // 128, 128), jnp.float32),
            pltpu.VMEM_SHARED((256,), jnp.float32),
            pltpu.SemaphoreType.DMA,
            pltpu.SemaphoreType.DMA,
        ],
    )
    agg, _unused_alpha = f(qh, kh, vh, srcv, dstv)
    return agg


PB = 768
NH = NP // 2


def _qkvs_body(h_ref, wq_ref, wk_ref, wv_ref, ws_ref,
               bq_ref, bk_ref, bv_ref, bs_ref,
               q_ref, k_ref, v_ref, s_ref):
    hb = h_ref[...]
    q_ref[...] = jnp.dot(hb, wq_ref[...],
                         preferred_element_type=jnp.float32) + bq_ref[...]
    k_ref[...] = jnp.dot(hb, wk_ref[...],
                         preferred_element_type=jnp.float32) + bk_ref[...]
    v_ref[...] = jnp.dot(hb, wv_ref[...],
                         preferred_element_type=jnp.float32) + bv_ref[...]
    s_ref[...] = jnp.dot(hb, ws_ref[...],
                         preferred_element_type=jnp.float32) + bs_ref[...]


def _qkvs(h, Wq, Wk, Wv, Ws, bq, bk, bv, bs):
    cin = h.shape[1]
    wspec = pl.BlockSpec((cin, D), lambda i: (0, 0))
    bspec = pl.BlockSpec((1, D), lambda i: (0, 0))
    ospec = pl.BlockSpec((512, D), lambda i: (i, 0))
    outs = pl.pallas_call(
        _qkvs_body,
        grid=(NP // 512,),
        in_specs=[pl.BlockSpec((512, cin), lambda i: (i, 0)),
                  wspec, wspec, wspec, wspec,
                  bspec, bspec, bspec, bspec],
        out_specs=[ospec, ospec, ospec, ospec],
        out_shape=[jax.ShapeDtypeStruct((NP, D), jnp.float32)] * 4,
    )(h, Wq, Wk, Wv, Ws, bq.reshape(1, D), bk.reshape(1, D),
      bv.reshape(1, D), bs.reshape(1, D))
    return outs


def _postln_body(a0_ref, a1_ref, s_ref, g_ref, b_ref, h_ref):
    xx = jnp.concatenate([a0_ref[...], a1_ref[...]], axis=1) + s_ref[...]
    xx = jax.nn.relu(xx)
    mu = jnp.mean(xx, axis=-1, keepdims=True)
    var = jnp.mean(jnp.square(xx - mu), axis=-1, keepdims=True)
    h_ref[...] = (xx - mu) / jnp.sqrt(var + EPS) * g_ref[...] + b_ref[...]


def _postln(agg, s, g, b):
    return pl.pallas_call(
        _postln_body,
        grid=(NP // 512,),
        in_specs=[pl.BlockSpec((512, OC), lambda i: (i, 0)),
                  pl.BlockSpec((512, OC), lambda i: (i + NP // 512, 0)),
                  pl.BlockSpec((512, D), lambda i: (i, 0)),
                  pl.BlockSpec((1, D), lambda i: (0, 0)),
                  pl.BlockSpec((1, D), lambda i: (0, 0))],
        out_specs=pl.BlockSpec((512, D), lambda i: (i, 0)),
        out_shape=jax.ShapeDtypeStruct((NP, D), jnp.float32),
    )(agg, agg, s, g.reshape(1, D), b.reshape(1, D))


def _tconv_sc(h, srcv, dstv, Wq, bq, Wk, bk, Wv, bv, Ws, bs, g, b):
    q, k, v, s = _qkvs(h, Wq, Wk, Wv, Ws, bq, bk, bv, bs)
    agg = _edge_call(q.reshape(2 * NP, OC), k.reshape(2 * NP, OC),
                     v.reshape(2 * NP, OC), srcv, dstv)
    return _postln(agg, s, g, b)


def _decoder_body(pooled_ref, gc_ref, sl_ref, fcWp_ref, fcg_ref,
                  fcs_ref, fcb_ref, emb_ref, Wih_ref, Whh_ref,
                  bihh_ref, outW_ref, outb_ref, out_ref):
    pooled = pooled_ref[...]
    enc = jax.nn.relu(
        jnp.dot(pooled, fcWp_ref[...], preferred_element_type=jnp.float32)
        + jnp.dot(gc_ref[...], fcg_ref[...],
                  preferred_element_type=jnp.float32)
        + jnp.dot(sl_ref[...], fcs_ref[...],
                  preferred_element_type=jnp.float32)
        + fcb_ref[...])
    emb = emb_ref[...]
    Wih = Wih_ref[...]
    Whh = Whh_ref[...]
    bihh = bihh_ref[...]
    outW = outW_ref[...]
    outb = outb_ref[...]
    hs = enc
    cs = jnp.zeros_like(enc)
    inp = jnp.broadcast_to(emb[1], (B, ED))
    for t in range(T):
        gates = (jnp.dot(inp, Wih, preferred_element_type=jnp.float32)
                 + jnp.dot(hs, Whh, preferred_element_type=jnp.float32)
                 + bihh)
        i_ = gates[:, 0 * ED:1 * ED]
        f_ = gates[:, 1 * ED:2 * ED]
        g_ = gates[:, 2 * ED:3 * ED]
        o_ = gates[:, 3 * ED:4 * ED]
        cs = jax.nn.sigmoid(f_) * cs + jax.nn.sigmoid(i_) * jnp.tanh(g_)
        hs = jax.nn.sigmoid(o_) * jnp.tanh(cs)
        logits = (jnp.dot(hs, outW, preferred_element_type=jnp.float32)
                  + outb)
        out_ref[:, t, :] = logits
        m = jnp.max(logits, axis=-1, keepdims=True)
        iota = lax.broadcasted_iota(jnp.int32, (B, V), 1)
        tok = jnp.min(jnp.where(logits == m, iota, V), axis=-1,
                      keepdims=True)
        onehot = (lax.broadcasted_iota(jnp.int32, (B, V), 1)
                  == tok).astype(jnp.float32)
        inp = jnp.dot(onehot, emb, preferred_element_type=jnp.float32)


def _decode(pooled, gc, sl, fcWp, fcg, fcs, fcb, emb, Wih, Whh, bihh,
            outW, outb):
    return pl.pallas_call(
        _decoder_body,
        out_shape=jax.ShapeDtypeStruct((B, T, V), jnp.float32),
    )(pooled, gc, sl, fcWp, fcg, fcs, fcb, emb, Wih, Whh, bihh,
      outW, outb)


def kernel(x, edge_index, batch, gc_content, seq_length,
           Wq0, bq0, Wk0, bk0, Wv0, bv0, Ws0, bs0, g0, b0,
           Wq1, bq1, Wk1, bk1, Wv1, bv1, Ws1, bs1, g1, b1,
           Wq2, bq2, Wk2, bk2, Wv2, bv2, Ws2, bs2, g2, b2,
           fcW, fcb, emb, Wih, Whh, bih, bhh, outW, outb):
    srcv = edge_index[0]
    dstv = edge_index[1]
    xp = jnp.zeros((NP, 4), jnp.float32).at[:N].set(x)
    batch_pad = jnp.concatenate(
        [batch, jnp.full((NP - N,), B, jnp.int32)])
    layers = [
        (Wq0, bq0, Wk0, bk0, Wv0, bv0, Ws0, bs0, g0, b0),
        (Wq1, bq1, Wk1, bk1, Wv1, bv1, Ws1, bs1, g1, b1),
        (Wq2, bq2, Wk2, bk2, Wv2, bv2, Ws2, bs2, g2, b2),
    ]
    h = xp
    for (Wq, bq, Wk, bk, Wv, bv, Ws, bs, g, b) in layers:
        h = _tconv_sc(h, srcv, dstv, Wq, bq, Wk, bk, Wv, bv, Ws, bs, g, b)
    hn = h[:N]
    cnt = jax.ops.segment_sum(jnp.ones((N,), jnp.float32), batch,
                              num_segments=B)
    pooled = (jax.ops.segment_sum(hn, batch, num_segments=B)
              / jnp.maximum(cnt, 1.0)[:, None])
    return _decode(pooled,
                   gc_content.reshape(B, 1), seq_length.reshape(B, 1),
                   fcW[:D], fcW[D:D + 1], fcW[D + 1:D + 2], fcb.reshape(1, ED),
                   emb, Wih, Whh, (bih + bhh).reshape(1, 4 * ED), outW,
                   outb.reshape(1, V))


# speedup vs baseline: 1.0587x; 1.0450x over previous
import functools

import jax
import jax.numpy as jnp
from jax import lax
from jax.experimental import pallas as pl
from jax.experimental.pallas import tpu as pltpu
from jax.experimental.pallas import tpu_sc as plsc

N = 10000
NP = 10240
E = 160000
B = 512
H = 2
OC = 128
D = H * OC
ED = 64
V = 5
T = 21
EPS = 1e-5

EP = E // 16
CH = 80
NCH = EP // CH
NPT = NP // 16
SCALE = 1.0 / (OC ** 0.5)


def _edge_body(qh, kh, vh, eph, out, alout,
               qidx, kidx, dstb, tmp160, arow, qbuf, kbuf, exrow,
               den2, dtmp2, iob, vmax16, mbuf, aggbuf,
               agg_s, den_s, maxes_s, sem0, sem1):
    c = lax.axis_index("c")
    w = lax.axis_index("s")
    ebase = w * EP
    nbase = w * NPT
    zero16 = jnp.zeros((16,), jnp.float32)
    izero = jnp.zeros((16,), jnp.int32)
    lane = lax.iota(jnp.int32, 16)

    dnums = lax.GatherDimensionNumbers(
        offset_dims=(), collapsed_slice_dims=(0,), start_index_map=(0,))

    def permute(vec, idxv):
        return lax.gather(vec, idxv[:, None], dnums, (1,),
                          mode=lax.GatherScatterMode.PROMISE_IN_BOUNDS)

    def bcast_lane(vec, r):
        return permute(vec, izero + r)

    bfly = [lane ^ k for k in (8, 4, 2, 1)]

    def lanesum(vec):
        for ix in bfly:
            vec = vec + permute(vec, ix)
        return vec

    def z_qbuf(r, _):
        for jj in range(8):
            qbuf[r, pl.ds(jj * 16, 16)] = zero16
        return 0
    lax.fori_loop(0, CH, z_qbuf, 0)
    for t in range(NPT // CH):
        pltpu.sync_copy(qbuf, agg_s.at[pl.ds(pl.multiple_of(nbase + t * CH, 16), CH)])

    def z_den(i, _):
        for jj in range(8):
            den2[i, pl.ds(jj * 16, 16)] = zero16
        return 0
    lax.fori_loop(0, NP // 128, z_den, 0)

    def z_dt(i, _):
        for jj in range(8):
            dtmp2[i, pl.ds(jj * 16, 16)] = zero16
        return 0
    lax.fori_loop(0, 5, z_dt, 0)
    pltpu.sync_copy(dtmp2, den_s.at[pl.ds(pl.multiple_of(w * 5, 1), 5)])
    for i in range(5):
        iob[0, pl.ds(i * 16, 16)] = i * 16 + lane

    def p1_chunk(j, runmax):
        eb = ebase + j * CH
        pltpu.sync_copy(eph.at[w * NCH + j], tmp160)
        for i in range(CH // 16):
            d16 = tmp160[pl.ds(i * 16, 16)]
            qidx[pl.ds(i * 16, 16)] = d16 * 2 + c
            s16 = tmp160[pl.ds(CH + i * 16, 16)]
            kidx[pl.ds(i * 16, 16)] = s16 * 2 + c

        cq = pltpu.async_copy(qh.at[qidx], qbuf, sem0)
        ck = pltpu.async_copy(kh.at[kidx], kbuf, sem1)
        cq.wait()
        ck.wait()

        def grp(g, rm):
            def sub(s4, a16):
                for u in range(4):
                    e16 = s4 * 4 + u
                    e = g * 16 + e16
                    acc = qbuf[e, pl.ds(0, 16)] * kbuf[e, pl.ds(0, 16)]
                    for jj in range(1, 8):
                        acc = acc + (qbuf[e, pl.ds(jj * 16, 16)]
                                     * kbuf[e, pl.ds(jj * 16, 16)])
                    aval = lanesum(acc)
                    a16 = jnp.where(lane == e16, aval, a16)
                return a16
            a16 = lax.fori_loop(0, 4, sub, zero16) * SCALE
            arow[pl.ds(g * 16, 16)] = a16
            return jnp.maximum(rm, a16)
        runmax = lax.fori_loop(0, CH // 16, grp, runmax)
        pltpu.sync_copy(arow, alout.at[pl.ds(pl.multiple_of(c * E + eb, 16), CH)])
        return runmax

    runmax = lax.fori_loop(0, NCH, p1_chunk,
                           jnp.full((16,), -1e30, jnp.float32))
    vmax16[pl.ds(0, 16)] = runmax
    pltpu.sync_copy(vmax16, maxes_s.at[pl.ds(pl.multiple_of(w * 16, 16), 16)])
    plsc.subcore_barrier()

    pltpu.sync_copy(maxes_s, mbuf)
    m = mbuf[pl.ds(0, 16)]
    for t in range(1, 16):
        m = jnp.maximum(m, mbuf[pl.ds(t * 16, 16)])
    gmax = jnp.max(m)

    def p2_chunk(j, _):
        eb = ebase + j * CH
        pltpu.sync_copy(eph.at[w * NCH + j], tmp160)
        for i in range(CH // 16):
            s16 = tmp160[pl.ds(CH + i * 16, 16)]
            kidx[pl.ds(i * 16, 16)] = s16 * 2 + c
        cv = pltpu.async_copy(vh.at[kidx], qbuf, sem0)
        for i in range(CH // 16):
            dstb[0, pl.ds(i * 16, 16)] = tmp160[pl.ds(i * 16, 16)]
        pltpu.sync_copy(alout.at[pl.ds(pl.multiple_of(c * E + eb, 16), CH)], arow)

        def exg(i, _):
            a16 = arow[pl.ds(i * 16, 16)]
            e16 = jnp.exp(a16 - gmax)
            exrow[pl.ds(i * 16, 16)] = e16
            d16 = dstb[0, pl.ds(i * 16, 16)]
            plsc.addupdate_scatter(den2, [d16 >> 7, d16 & 127], e16)
            return 0
        lax.fori_loop(0, CH // 16, exg, 0)
        cv.wait()

        def sgrp(g, _):
            ex16 = exrow[pl.ds(g * 16, 16)]

            def ssub(s4, _):
                for u in range(4):
                    r = s4 * 4 + u
                    e = g * 16 + r
                    bc = bcast_lane(ex16, r)
                    for jj in range(8):
                        qbuf[e, pl.ds(jj * 16, 16)] = (
                            qbuf[e, pl.ds(jj * 16, 16)] * bc)
                return 0
            lax.fori_loop(0, 4, ssub, 0)
            return 0
        lax.fori_loop(0, CH // 16, sgrp, 0)
        pltpu.sync_copy(qbuf, agg_s.at[dstb.at[0]], add=True)
        return 0
    lax.fori_loop(0, NCH, p2_chunk, 0)
    pltpu.sync_copy(den2, den_s.at[iob.at[0]], add=True)
    plsc.subcore_barrier()

    pltpu.sync_copy(den_s.at[pl.ds(pl.multiple_of(w * 5, 1), 5)], dtmp2)

    def outc(ch, _):
        rb = nbase + ch * 16
        pltpu.sync_copy(agg_s.at[pl.ds(pl.multiple_of(rb, 16), 16)], aggbuf)

        inv16 = 1.0 / (dtmp2[ch >> 3, pl.ds((ch & 7) * 16, 16)] + 1e-16)

        def rown(r, _):
            bc = bcast_lane(inv16, r)
            for jj in range(8):
                aggbuf[r, pl.ds(jj * 16, 16)] = (
                    aggbuf[r, pl.ds(jj * 16, 16)] * bc)
            return 0
        lax.fori_loop(0, 16, rown, 0)
        pltpu.sync_copy(aggbuf, out.at[pl.ds(pl.multiple_of(c * NP + rb, 16), 16)])
        return 0
    lax.fori_loop(0, NPT // 16, outc, 0)


def _edge_call(qh, kh, vh, ep):
    mesh = plsc.VectorSubcoreMesh(core_axis_name="c", subcore_axis_name="s")
    f = pl.kernel(
        _edge_body,
        out_type=(jax.ShapeDtypeStruct((H * NP, OC), jnp.float32),
                  jax.ShapeDtypeStruct((H * E,), jnp.float32)),
        mesh=mesh,
        compiler_params=pltpu.CompilerParams(needs_layout_passes=False),
        scratch_types=[
            pltpu.VMEM((CH,), jnp.int32),
            pltpu.VMEM((CH,), jnp.int32),
            pltpu.VMEM((1, CH), jnp.int32),
            pltpu.VMEM((2 * CH,), jnp.int32),
            pltpu.VMEM((CH,), jnp.float32),
            pltpu.VMEM((CH, OC), jnp.float32),
            pltpu.VMEM((CH, OC), jnp.float32),
            pltpu.VMEM((CH,), jnp.float32),
            pltpu.VMEM((NP // 128, 128), jnp.float32),
            pltpu.VMEM((5, 128), jnp.float32),
            pltpu.VMEM((1, 80), jnp.int32),
            pltpu.VMEM((16,), jnp.float32),
            pltpu.VMEM((256,), jnp.float32),
            pltpu.VMEM((16, OC), jnp.float32),
            pltpu.VMEM_SHARED((NP, OC), jnp.float32),
            pltpu.VMEM_SHARED((NP // 128, 128), jnp.float32),
            pltpu.VMEM_SHARED((256,), jnp.float32),
            pltpu.SemaphoreType.DMA,
            pltpu.SemaphoreType.DMA,
        ],
    )
    agg, _unused_alpha = f(qh, kh, vh, ep)
    return agg


PB = 768
NH = NP // 2


def _qkvs_body(h_ref, wq_ref, wk_ref, wv_ref, ws_ref,
               bq_ref, bk_ref, bv_ref, bs_ref,
               q_ref, k_ref, v_ref, s_ref):
    hb = h_ref[...]
    q_ref[...] = jnp.dot(hb, wq_ref[...],
                         preferred_element_type=jnp.float32) + bq_ref[...]
    k_ref[...] = jnp.dot(hb, wk_ref[...],
                         preferred_element_type=jnp.float32) + bk_ref[...]
    v_ref[...] = jnp.dot(hb, wv_ref[...],
                         preferred_element_type=jnp.float32) + bv_ref[...]
    s_ref[...] = jnp.dot(hb, ws_ref[...],
                         preferred_element_type=jnp.float32) + bs_ref[...]


def _qkvs(h, Wq, Wk, Wv, Ws, bq, bk, bv, bs):
    cin = h.shape[1]
    wspec = pl.BlockSpec((cin, D), lambda i: (0, 0))
    bspec = pl.BlockSpec((1, D), lambda i: (0, 0))
    ospec = pl.BlockSpec((512, D), lambda i: (i, 0))
    outs = pl.pallas_call(
        _qkvs_body,
        grid=(NP // 512,),
        in_specs=[pl.BlockSpec((512, cin), lambda i: (i, 0)),
                  wspec, wspec, wspec, wspec,
                  bspec, bspec, bspec, bspec],
        out_specs=[ospec, ospec, ospec, ospec],
        out_shape=[jax.ShapeDtypeStruct((NP, D), jnp.float32)] * 4,
    )(h, Wq, Wk, Wv, Ws, bq.reshape(1, D), bk.reshape(1, D),
      bv.reshape(1, D), bs.reshape(1, D))
    return outs


def _postln_body(a0_ref, a1_ref, s_ref, g_ref, b_ref, h_ref):
    xx = jnp.concatenate([a0_ref[...], a1_ref[...]], axis=1) + s_ref[...]
    xx = jax.nn.relu(xx)
    mu = jnp.mean(xx, axis=-1, keepdims=True)
    var = jnp.mean(jnp.square(xx - mu), axis=-1, keepdims=True)
    h_ref[...] = (xx - mu) / jnp.sqrt(var + EPS) * g_ref[...] + b_ref[...]


def _postln(agg, s, g, b):
    return pl.pallas_call(
        _postln_body,
        grid=(NP // 512,),
        in_specs=[pl.BlockSpec((512, OC), lambda i: (i, 0)),
                  pl.BlockSpec((512, OC), lambda i: (i + NP // 512, 0)),
                  pl.BlockSpec((512, D), lambda i: (i, 0)),
                  pl.BlockSpec((1, D), lambda i: (0, 0)),
                  pl.BlockSpec((1, D), lambda i: (0, 0))],
        out_specs=pl.BlockSpec((512, D), lambda i: (i, 0)),
        out_shape=jax.ShapeDtypeStruct((NP, D), jnp.float32),
    )(agg, agg, s, g.reshape(1, D), b.reshape(1, D))


def _tconv_sc(h, ep, Wq, bq, Wk, bk, Wv, bv, Ws, bs, g, b):
    q, k, v, s = _qkvs(h, Wq, Wk, Wv, Ws, bq, bk, bv, bs)
    agg = _edge_call(q.reshape(2 * NP, OC), k.reshape(2 * NP, OC),
                     v.reshape(2 * NP, OC), ep)
    return _postln(agg, s, g, b)


def _decoder_body(pooled_ref, gc_ref, sl_ref, fcWp_ref, fcg_ref,
                  fcs_ref, fcb_ref, emb_ref, Wih_ref, Whh_ref,
                  bihh_ref, outW_ref, outb_ref, out_ref):
    pooled = pooled_ref[...]
    enc = jax.nn.relu(
        jnp.dot(pooled, fcWp_ref[...], preferred_element_type=jnp.float32)
        + jnp.dot(gc_ref[...], fcg_ref[...],
                  preferred_element_type=jnp.float32)
        + jnp.dot(sl_ref[...], fcs_ref[...],
                  preferred_element_type=jnp.float32)
        + fcb_ref[...])
    emb = emb_ref[...]
    Wih = Wih_ref[...]
    Whh = Whh_ref[...]
    bihh = bihh_ref[...]
    outW = outW_ref[...]
    outb = outb_ref[...]
    hs = enc
    cs = jnp.zeros_like(enc)
    inp = jnp.broadcast_to(emb[1], (B, ED))
    for t in range(T):
        gates = (jnp.dot(inp, Wih, preferred_element_type=jnp.float32)
                 + jnp.dot(hs, Whh, preferred_element_type=jnp.float32)
                 + bihh)
        i_ = gates[:, 0 * ED:1 * ED]
        f_ = gates[:, 1 * ED:2 * ED]
        g_ = gates[:, 2 * ED:3 * ED]
        o_ = gates[:, 3 * ED:4 * ED]
        cs = jax.nn.sigmoid(f_) * cs + jax.nn.sigmoid(i_) * jnp.tanh(g_)
        hs = jax.nn.sigmoid(o_) * jnp.tanh(cs)
        logits = (jnp.dot(hs, outW, preferred_element_type=jnp.float32)
                  + outb)
        out_ref[:, t, :] = logits
        m = jnp.max(logits, axis=-1, keepdims=True)
        iota = lax.broadcasted_iota(jnp.int32, (B, V), 1)
        tok = jnp.min(jnp.where(logits == m, iota, V), axis=-1,
                      keepdims=True)
        onehot = (lax.broadcasted_iota(jnp.int32, (B, V), 1)
                  == tok).astype(jnp.float32)
        inp = jnp.dot(onehot, emb, preferred_element_type=jnp.float32)


def _decode(pooled, gc, sl, fcWp, fcg, fcs, fcb, emb, Wih, Whh, bihh,
            outW, outb):
    return pl.pallas_call(
        _decoder_body,
        out_shape=jax.ShapeDtypeStruct((B, T, V), jnp.float32),
    )(pooled, gc, sl, fcWp, fcg, fcs, fcb, emb, Wih, Whh, bihh,
      outW, outb)


def kernel(x, edge_index, batch, gc_content, seq_length,
           Wq0, bq0, Wk0, bk0, Wv0, bv0, Ws0, bs0, g0, b0,
           Wq1, bq1, Wk1, bk1, Wv1, bv1, Ws1, bs1, g1, b1,
           Wq2, bq2, Wk2, bk2, Wv2, bv2, Ws2, bs2, g2, b2,
           fcW, fcb, emb, Wih, Whh, bih, bhh, outW, outb):
    srcv = edge_index[0]
    dstv = edge_index[1]
    ep = jnp.concatenate([dstv.reshape(E // CH, CH),
                          srcv.reshape(E // CH, CH)], axis=1)
    xp = jnp.zeros((NP, 4), jnp.float32).at[:N].set(x)
    batch_pad = jnp.concatenate(
        [batch, jnp.full((NP - N,), B, jnp.int32)])
    layers = [
        (Wq0, bq0, Wk0, bk0, Wv0, bv0, Ws0, bs0, g0, b0),
        (Wq1, bq1, Wk1, bk1, Wv1, bv1, Ws1, bs1, g1, b1),
        (Wq2, bq2, Wk2, bk2, Wv2, bv2, Ws2, bs2, g2, b2),
    ]
    h = xp
    for (Wq, bq, Wk, bk, Wv, bv, Ws, bs, g, b) in layers:
        h = _tconv_sc(h, ep, Wq, bq, Wk, bk, Wv, bv, Ws, bs, g, b)
    hn = h[:N]
    cnt = jax.ops.segment_sum(jnp.ones((N,), jnp.float32), batch,
                              num_segments=B)
    pooled = (jax.ops.segment_sum(hn, batch, num_segments=B)
              / jnp.maximum(cnt, 1.0)[:, None])
    return _decode(pooled,
                   gc_content.reshape(B, 1), seq_length.reshape(B, 1),
                   fcW[:D], fcW[D:D + 1], fcW[D + 1:D + 2], fcb.reshape(1, ED),
                   emb, Wih, Whh, (bih + bhh).reshape(1, 4 * ED), outW,
                   outb.reshape(1, V))


# speedup vs baseline: 1.0672x; 1.0080x over previous
import functools

import jax
import jax.numpy as jnp
from jax import lax
from jax.experimental import pallas as pl
from jax.experimental.pallas import tpu as pltpu
from jax.experimental.pallas import tpu_sc as plsc

N = 10000
NP = 10240
E = 160000
B = 512
H = 2
OC = 128
D = H * OC
ED = 64
V = 5
T = 21
EPS = 1e-5

EP = E // 16
CH = 80
NCH = EP // CH
NPT = NP // 16
SCALE = 1.0 / (OC ** 0.5)


def _edge_body(qh, kh, vh, eph, out, alout,
               qidx, kidx, dstb, tmp160, arow, qbuf, kbuf, exrow,
               den2, dtmp2, iob, vmax16, mbuf, aggbuf,
               agg_s, den_s, maxes_s, sem0, sem1, sem2, sem3):
    c = lax.axis_index("c")
    w = lax.axis_index("s")
    ebase = w * EP
    nbase = w * NPT
    zero16 = jnp.zeros((16,), jnp.float32)
    izero = jnp.zeros((16,), jnp.int32)
    lane = lax.iota(jnp.int32, 16)

    dnums = lax.GatherDimensionNumbers(
        offset_dims=(), collapsed_slice_dims=(0,), start_index_map=(0,))

    def permute(vec, idxv):
        return lax.gather(vec, idxv[:, None], dnums, (1,),
                          mode=lax.GatherScatterMode.PROMISE_IN_BOUNDS)

    def bcast_lane(vec, r):
        return permute(vec, izero + r)

    bfly = [lane ^ k for k in (8, 4, 2, 1)]

    def lanesum(vec):
        for ix in bfly:
            vec = vec + permute(vec, ix)
        return vec

    def z_qbuf(r, _):
        for jj in range(8):
            qbuf[r, pl.ds(jj * 16, 16)] = zero16
        return 0
    lax.fori_loop(0, CH, z_qbuf, 0)
    for t in range(NPT // CH):
        pltpu.sync_copy(qbuf, agg_s.at[pl.ds(pl.multiple_of(nbase + t * CH, 16), CH)])

    def z_den(i, _):
        for jj in range(8):
            den2[i, pl.ds(jj * 16, 16)] = zero16
        return 0
    lax.fori_loop(0, NP // 128, z_den, 0)

    def z_dt(i, _):
        for jj in range(8):
            dtmp2[i, pl.ds(jj * 16, 16)] = zero16
        return 0
    lax.fori_loop(0, 5, z_dt, 0)
    pltpu.sync_copy(dtmp2, den_s.at[pl.ds(pl.multiple_of(w * 5, 1), 5)])
    for i in range(5):
        iob[0, pl.ds(i * 16, 16)] = i * 16 + lane

    def p1_chunk(j, runmax):
        eb = ebase + j * CH
        pltpu.sync_copy(eph.at[w * NCH + j], tmp160)
        for i in range(CH // 16):
            d16 = tmp160[pl.ds(i * 16, 16)]
            qidx[pl.ds(i * 16, 16)] = d16 * 2 + c
            s16 = tmp160[pl.ds(CH + i * 16, 16)]
            kidx[pl.ds(i * 16, 16)] = s16 * 2 + c

        cq0 = pltpu.async_copy(qh.at[qidx.at[pl.ds(0, 48)]],
                               qbuf.at[pl.ds(0, 48)], sem0)
        ck0 = pltpu.async_copy(kh.at[kidx.at[pl.ds(0, 48)]],
                               kbuf.at[pl.ds(0, 48)], sem1)
        cq1 = pltpu.async_copy(qh.at[qidx.at[pl.ds(48, 32)]],
                               qbuf.at[pl.ds(48, 32)], sem2)
        ck1 = pltpu.async_copy(kh.at[kidx.at[pl.ds(48, 32)]],
                               kbuf.at[pl.ds(48, 32)], sem3)
        cq0.wait()
        ck0.wait()

        def grp(g, rm):
            def sub(s4, a16):
                for u in range(4):
                    e16 = s4 * 4 + u
                    e = g * 16 + e16
                    acc = qbuf[e, pl.ds(0, 16)] * kbuf[e, pl.ds(0, 16)]
                    for jj in range(1, 8):
                        acc = acc + (qbuf[e, pl.ds(jj * 16, 16)]
                                     * kbuf[e, pl.ds(jj * 16, 16)])
                    aval = lanesum(acc)
                    a16 = jnp.where(lane == e16, aval, a16)
                return a16
            a16 = lax.fori_loop(0, 4, sub, zero16) * SCALE
            arow[pl.ds(g * 16, 16)] = a16
            return jnp.maximum(rm, a16)
        runmax = lax.fori_loop(0, 3, grp, runmax)
        cq1.wait()
        ck1.wait()
        runmax = lax.fori_loop(3, CH // 16, grp, runmax)
        pltpu.sync_copy(arow, alout.at[pl.ds(pl.multiple_of(c * E + eb, 16), CH)])
        return runmax

    runmax = lax.fori_loop(0, NCH, p1_chunk,
                           jnp.full((16,), -1e30, jnp.float32))
    vmax16[pl.ds(0, 16)] = runmax
    pltpu.sync_copy(vmax16, maxes_s.at[pl.ds(pl.multiple_of(w * 16, 16), 16)])
    plsc.subcore_barrier()

    pltpu.sync_copy(maxes_s, mbuf)
    m = mbuf[pl.ds(0, 16)]
    for t in range(1, 16):
        m = jnp.maximum(m, mbuf[pl.ds(t * 16, 16)])
    gmax = jnp.max(m)

    def p2_chunk(j, _):
        eb = ebase + j * CH
        pltpu.sync_copy(eph.at[w * NCH + j], tmp160)
        for i in range(CH // 16):
            s16 = tmp160[pl.ds(CH + i * 16, 16)]
            kidx[pl.ds(i * 16, 16)] = s16 * 2 + c
        cv0 = pltpu.async_copy(vh.at[kidx.at[pl.ds(0, 48)]],
                               qbuf.at[pl.ds(0, 48)], sem0)
        cv1 = pltpu.async_copy(vh.at[kidx.at[pl.ds(48, 32)]],
                               qbuf.at[pl.ds(48, 32)], sem1)
        for i in range(CH // 16):
            dstb[0, pl.ds(i * 16, 16)] = tmp160[pl.ds(i * 16, 16)]
        pltpu.sync_copy(alout.at[pl.ds(pl.multiple_of(c * E + eb, 16), CH)], arow)

        def exg(i, _):
            a16 = arow[pl.ds(i * 16, 16)]
            e16 = jnp.exp(a16 - gmax)
            exrow[pl.ds(i * 16, 16)] = e16
            d16 = dstb[0, pl.ds(i * 16, 16)]
            plsc.addupdate_scatter(den2, [d16 >> 7, d16 & 127], e16)
            return 0
        lax.fori_loop(0, CH // 16, exg, 0)
        cv0.wait()

        def sgrp(g, _):
            ex16 = exrow[pl.ds(g * 16, 16)]

            def ssub(s4, _):
                for u in range(4):
                    r = s4 * 4 + u
                    e = g * 16 + r
                    bc = bcast_lane(ex16, r)
                    for jj in range(8):
                        qbuf[e, pl.ds(jj * 16, 16)] = (
                            qbuf[e, pl.ds(jj * 16, 16)] * bc)
                return 0
            lax.fori_loop(0, 4, ssub, 0)
            return 0
        lax.fori_loop(0, 3, sgrp, 0)
        cv1.wait()
        lax.fori_loop(3, CH // 16, sgrp, 0)
        pltpu.sync_copy(qbuf, agg_s.at[dstb.at[0]], add=True)
        return 0
    lax.fori_loop(0, NCH, p2_chunk, 0)
    pltpu.sync_copy(den2, den_s.at[iob.at[0]], add=True)
    plsc.subcore_barrier()

    pltpu.sync_copy(den_s.at[pl.ds(pl.multiple_of(w * 5, 1), 5)], dtmp2)

    def outc(ch, _):
        rb = nbase + ch * 16
        pltpu.sync_copy(agg_s.at[pl.ds(pl.multiple_of(rb, 16), 16)], aggbuf)

        inv16 = 1.0 / (dtmp2[ch >> 3, pl.ds((ch & 7) * 16, 16)] + 1e-16)

        def rown(r, _):
            bc = bcast_lane(inv16, r)
            for jj in range(8):
                aggbuf[r, pl.ds(jj * 16, 16)] = (
                    aggbuf[r, pl.ds(jj * 16, 16)] * bc)
            return 0
        lax.fori_loop(0, 16, rown, 0)
        pltpu.sync_copy(aggbuf, out.at[pl.ds(pl.multiple_of(c * NP + rb, 16), 16)])
        return 0
    lax.fori_loop(0, NPT // 16, outc, 0)


def _edge_call(qh, kh, vh, ep):
    mesh = plsc.VectorSubcoreMesh(core_axis_name="c", subcore_axis_name="s")
    f = pl.kernel(
        _edge_body,
        out_type=(jax.ShapeDtypeStruct((H * NP, OC), jnp.float32),
                  jax.ShapeDtypeStruct((H * E,), jnp.float32)),
        mesh=mesh,
        compiler_params=pltpu.CompilerParams(needs_layout_passes=False),
        scratch_types=[
            pltpu.VMEM((CH,), jnp.int32),
            pltpu.VMEM((CH,), jnp.int32),
            pltpu.VMEM((1, CH), jnp.int32),
            pltpu.VMEM((2 * CH,), jnp.int32),
            pltpu.VMEM((CH,), jnp.float32),
            pltpu.VMEM((CH, OC), jnp.float32),
            pltpu.VMEM((CH, OC), jnp.float32),
            pltpu.VMEM((CH,), jnp.float32),
            pltpu.VMEM((NP // 128, 128), jnp.float32),
            pltpu.VMEM((5, 128), jnp.float32),
            pltpu.VMEM((1, 80), jnp.int32),
            pltpu.VMEM((16,), jnp.float32),
            pltpu.VMEM((256,), jnp.float32),
            pltpu.VMEM((16, OC), jnp.float32),
            pltpu.VMEM_SHARED((NP, OC), jnp.float32),
            pltpu.VMEM_SHARED((NP // 128, 128), jnp.float32),
            pltpu.VMEM_SHARED((256,), jnp.float32),
            pltpu.SemaphoreType.DMA,
            pltpu.SemaphoreType.DMA,
            pltpu.SemaphoreType.DMA,
            pltpu.SemaphoreType.DMA,
        ],
    )
    agg, _unused_alpha = f(qh, kh, vh, ep)
    return agg


PB = 768
NH = NP // 2


def _qkvs_body(h_ref, wq_ref, wk_ref, wv_ref, ws_ref,
               bq_ref, bk_ref, bv_ref, bs_ref,
               q_ref, k_ref, v_ref, s_ref):
    hb = h_ref[...]
    q_ref[...] = jnp.dot(hb, wq_ref[...],
                         preferred_element_type=jnp.float32) + bq_ref[...]
    k_ref[...] = jnp.dot(hb, wk_ref[...],
                         preferred_element_type=jnp.float32) + bk_ref[...]
    v_ref[...] = jnp.dot(hb, wv_ref[...],
                         preferred_element_type=jnp.float32) + bv_ref[...]
    s_ref[...] = jnp.dot(hb, ws_ref[...],
                         preferred_element_type=jnp.float32) + bs_ref[...]


def _qkvs(h, Wq, Wk, Wv, Ws, bq, bk, bv, bs):
    cin = h.shape[1]
    wspec = pl.BlockSpec((cin, D), lambda i: (0, 0))
    bspec = pl.BlockSpec((1, D), lambda i: (0, 0))
    ospec = pl.BlockSpec((512, D), lambda i: (i, 0))
    outs = pl.pallas_call(
        _qkvs_body,
        grid=(NP // 512,),
        in_specs=[pl.BlockSpec((512, cin), lambda i: (i, 0)),
                  wspec, wspec, wspec, wspec,
                  bspec, bspec, bspec, bspec],
        out_specs=[ospec, ospec, ospec, ospec],
        out_shape=[jax.ShapeDtypeStruct((NP, D), jnp.float32)] * 4,
    )(h, Wq, Wk, Wv, Ws, bq.reshape(1, D), bk.reshape(1, D),
      bv.reshape(1, D), bs.reshape(1, D))
    return outs


def _postln_body(a0_ref, a1_ref, s_ref, g_ref, b_ref, h_ref):
    xx = jnp.concatenate([a0_ref[...], a1_ref[...]], axis=1) + s_ref[...]
    xx = jax.nn.relu(xx)
    mu = jnp.mean(xx, axis=-1, keepdims=True)
    var = jnp.mean(jnp.square(xx - mu), axis=-1, keepdims=True)
    h_ref[...] = (xx - mu) / jnp.sqrt(var + EPS) * g_ref[...] + b_ref[...]


def _postln(agg, s, g, b):
    return pl.pallas_call(
        _postln_body,
        grid=(NP // 512,),
        in_specs=[pl.BlockSpec((512, OC), lambda i: (i, 0)),
                  pl.BlockSpec((512, OC), lambda i: (i + NP // 512, 0)),
                  pl.BlockSpec((512, D), lambda i: (i, 0)),
                  pl.BlockSpec((1, D), lambda i: (0, 0)),
                  pl.BlockSpec((1, D), lambda i: (0, 0))],
        out_specs=pl.BlockSpec((512, D), lambda i: (i, 0)),
        out_shape=jax.ShapeDtypeStruct((NP, D), jnp.float32),
    )(agg, agg, s, g.reshape(1, D), b.reshape(1, D))


def _tconv_sc(h, ep, Wq, bq, Wk, bk, Wv, bv, Ws, bs, g, b):
    q, k, v, s = _qkvs(h, Wq, Wk, Wv, Ws, bq, bk, bv, bs)
    agg = _edge_call(q.reshape(2 * NP, OC), k.reshape(2 * NP, OC),
                     v.reshape(2 * NP, OC), ep)
    return _postln(agg, s, g, b)


def _decoder_body(pooled_ref, gc_ref, sl_ref, fcWp_ref, fcg_ref,
                  fcs_ref, fcb_ref, emb_ref, Wih_ref, Whh_ref,
                  bihh_ref, outW_ref, outb_ref, out_ref):
    pooled = pooled_ref[...]
    enc = jax.nn.relu(
        jnp.dot(pooled, fcWp_ref[...], preferred_element_type=jnp.float32)
        + jnp.dot(gc_ref[...], fcg_ref[...],
                  preferred_element_type=jnp.float32)
        + jnp.dot(sl_ref[...], fcs_ref[...],
                  preferred_element_type=jnp.float32)
        + fcb_ref[...])
    emb = emb_ref[...]
    Wih = Wih_ref[...]
    Whh = Whh_ref[...]
    bihh = bihh_ref[...]
    outW = outW_ref[...]
    outb = outb_ref[...]
    hs = enc
    cs = jnp.zeros_like(enc)
    inp = jnp.broadcast_to(emb[1], (B, ED))
    for t in range(T):
        gates = (jnp.dot(inp, Wih, preferred_element_type=jnp.float32)
                 + jnp.dot(hs, Whh, preferred_element_type=jnp.float32)
                 + bihh)
        i_ = gates[:, 0 * ED:1 * ED]
        f_ = gates[:, 1 * ED:2 * ED]
        g_ = gates[:, 2 * ED:3 * ED]
        o_ = gates[:, 3 * ED:4 * ED]
        cs = jax.nn.sigmoid(f_) * cs + jax.nn.sigmoid(i_) * jnp.tanh(g_)
        hs = jax.nn.sigmoid(o_) * jnp.tanh(cs)
        logits = (jnp.dot(hs, outW, preferred_element_type=jnp.float32)
                  + outb)
        out_ref[:, t, :] = logits
        m = jnp.max(logits, axis=-1, keepdims=True)
        iota = lax.broadcasted_iota(jnp.int32, (B, V), 1)
        tok = jnp.min(jnp.where(logits == m, iota, V), axis=-1,
                      keepdims=True)
        onehot = (lax.broadcasted_iota(jnp.int32, (B, V), 1)
                  == tok).astype(jnp.float32)
        inp = jnp.dot(onehot, emb, preferred_element_type=jnp.float32)


def _decode(pooled, gc, sl, fcWp, fcg, fcs, fcb, emb, Wih, Whh, bihh,
            outW, outb):
    return pl.pallas_call(
        _decoder_body,
        out_shape=jax.ShapeDtypeStruct((B, T, V), jnp.float32),
    )(pooled, gc, sl, fcWp, fcg, fcs, fcb, emb, Wih, Whh, bihh,
      outW, outb)


def kernel(x, edge_index, batch, gc_content, seq_length,
           Wq0, bq0, Wk0, bk0, Wv0, bv0, Ws0, bs0, g0, b0,
           Wq1, bq1, Wk1, bk1, Wv1, bv1, Ws1, bs1, g1, b1,
           Wq2, bq2, Wk2, bk2, Wv2, bv2, Ws2, bs2, g2, b2,
           fcW, fcb, emb, Wih, Whh, bih, bhh, outW, outb):
    srcv = edge_index[0]
    dstv = edge_index[1]
    ep = jnp.concatenate([dstv.reshape(E // CH, CH),
                          srcv.reshape(E // CH, CH)], axis=1)
    xp = jnp.zeros((NP, 4), jnp.float32).at[:N].set(x)
    batch_pad = jnp.concatenate(
        [batch, jnp.full((NP - N,), B, jnp.int32)])
    layers = [
        (Wq0, bq0, Wk0, bk0, Wv0, bv0, Ws0, bs0, g0, b0),
        (Wq1, bq1, Wk1, bk1, Wv1, bv1, Ws1, bs1, g1, b1),
        (Wq2, bq2, Wk2, bk2, Wv2, bv2, Ws2, bs2, g2, b2),
    ]
    h = xp
    for (Wq, bq, Wk, bk, Wv, bv, Ws, bs, g, b) in layers:
        h = _tconv_sc(h, ep, Wq, bq, Wk, bk, Wv, bv, Ws, bs, g, b)
    hn = h[:N]
    cnt = jax.ops.segment_sum(jnp.ones((N,), jnp.float32), batch,
                              num_segments=B)
    pooled = (jax.ops.segment_sum(hn, batch, num_segments=B)
              / jnp.maximum(cnt, 1.0)[:, None])
    return _decode(pooled,
                   gc_content.reshape(B, 1), seq_length.reshape(B, 1),
                   fcW[:D], fcW[D:D + 1], fcW[D + 1:D + 2], fcb.reshape(1, ED),
                   emb, Wih, Whh, (bih + bhh).reshape(1, 4 * ED), outW,
                   outb.reshape(1, V))
